# baseline (speedup 1.0000x reference)
import functools

import jax
import jax.numpy as jnp
from jax import lax
from jax.experimental import pallas as pl
from jax.experimental.pallas import tpu as pltpu
from jax.experimental.pallas import tpu_sc as plsc

N = 10000
E = 320000
D = 128
DOUT = 16

NC = 2
NS = 16
NW = NC * NS
CHUNK = 128
TNCH = 80
EPT = TNCH * CHUNK
EPAD = NW * EPT
NR = 79
NPAD = NR * 128
STRIPE = NPAD // NS
DSTRIPE = 2 * NPAD // NS

def _deg_kernel_body(didx_hbm, ones_hbm, zeros_hbm, out_hbm, idx_v, ones_v, buf_v, acc_sh):
    c = lax.axis_index("c")
    s = lax.axis_index("s")
    wid = s * NC + c
    pltpu.sync_copy(didx_hbm.at[wid], idx_v)
    stripe = pl.ds(s * DSTRIPE, DSTRIPE)
    pltpu.sync_copy(zeros_hbm.at[stripe], buf_v)
    pltpu.sync_copy(buf_v, acc_sh.at[stripe])
    plsc.subcore_barrier()

    pltpu.sync_copy(ones_hbm, ones_v)

    @pl.loop(0, 2 * TNCH)
    def _(j):
        pltpu.sync_copy(ones_v, acc_sh.at[idx_v.at[j]], add=True)

    plsc.subcore_barrier()
    pltpu.sync_copy(acc_sh.at[stripe], buf_v)
    pltpu.sync_copy(buf_v, out_hbm.at[pl.ds(c * 2 * NPAD + s * DSTRIPE, DSTRIPE)])


NBP = 8
CNT0 = 32
CNT1 = 128
TCHT = EPAD // CHUNK
C0TOT = NS * CNT0


def _ring(u_hbm, echunks_hbm, acc_sh, rows_v, slot_v, isem, gsem, ssem, start, cnt):
    for b in range(NBP):
        pltpu.async_copy(echunks_hbm.at[start + b], slot_v.at[b], isem[b])

    @pl.loop(0, cnt // NBP)
    def _(g):
        j0 = start + g * NBP
        for b in range(NBP):
            j = j0 + b
            pltpu.make_async_copy(echunks_hbm.at[j], slot_v.at[b], isem[b]).wait()
            pltpu.async_copy(u_hbm.at[slot_v.at[b, 0]], rows_v.at[b], gsem[b])
        for b in range(NBP):
            j = j0 + b
            pltpu.make_async_copy(u_hbm.at[slot_v.at[b, 0]], rows_v.at[b], gsem[b]).wait()
            pltpu.async_copy(rows_v.at[b], acc_sh.at[slot_v.at[b, 1]], ssem[b], add=True)
        for b in range(NBP):
            j = j0 + b
            jn = j + NBP
            pltpu.make_async_copy(rows_v.at[b], acc_sh.at[slot_v.at[b, 1]], ssem[b]).wait()

            @pl.when(jn < start + cnt)
            def _():
                pltpu.async_copy(echunks_hbm.at[jn], slot_v.at[b], isem[b])


def _aggp_body(nsplit, dc, *refs):
    us = refs[:nsplit]
    echunks_hbm = refs[nsplit]
    zeros_hbm = refs[nsplit + 1]
    outs = refs[nsplit + 2 : 2 * nsplit + 2]
    slot_v, rows_v, acc_sh = refs[2 * nsplit + 2 : 2 * nsplit + 5]
    sems = refs[2 * nsplit + 5 :]
    isem = sems[:NBP]
    gsem = sems[NBP : 2 * NBP]
    ssem = sems[2 * NBP :]
    c = lax.axis_index("c")
    s = lax.axis_index("s")
    stripe = pl.ds(s * STRIPE, STRIPE)

    for p in range(nsplit):
        u_hbm = us[p]
        pltpu.sync_copy(zeros_hbm.at[stripe], acc_sh.at[stripe])
        plsc.subcore_barrier()

        @pl.when(c == 0)
        def _():
            _ring(u_hbm, echunks_hbm, acc_sh, rows_v, slot_v,
                  isem, gsem, ssem, s * CNT0, CNT0)

        @pl.when(c == 1)
        def _():
            _ring(u_hbm, echunks_hbm, acc_sh, rows_v, slot_v,
                  isem, gsem, ssem, C0TOT + s * CNT1, CNT1)

        plsc.subcore_barrier()
        pltpu.sync_copy(acc_sh.at[stripe], outs[p].at[c, stripe])
        if p + 1 < nsplit:
            plsc.subcore_barrier()


@functools.lru_cache(maxsize=None)
def _sc_kernels(interpret=False):
    mesh = plsc.VectorSubcoreMesh(
        core_axis_name="c", subcore_axis_name="s", num_cores=NC, num_subcores=NS
    )
    deg = pl.kernel(
        _deg_kernel_body,
        out_type=jax.ShapeDtypeStruct((NC * 2 * NPAD,), jnp.float32),
        mesh=mesh,
        interpret=interpret,
        scratch_types=[
            pltpu.VMEM((2 * TNCH, CHUNK), jnp.int32),
            pltpu.VMEM((CHUNK,), jnp.float32),
            pltpu.VMEM((DSTRIPE,), jnp.float32),
            pltpu.VMEM_SHARED((2 * NPAD,), jnp.float32),
        ],
    )

    def make_agg(nsplit, dc):
        return pl.kernel(
            functools.partial(_aggp_body, nsplit, dc),
            out_type=[jax.ShapeDtypeStruct((NC, NPAD, dc), jnp.float32)] * nsplit,
            mesh=mesh,
            interpret=interpret,
            compiler_params=pltpu.CompilerParams(use_tc_tiling_on_sc=False),
            scratch_types=[
                pltpu.VMEM((NBP, 2, CHUNK), jnp.int32),
                pltpu.VMEM((NBP, CHUNK, dc), jnp.float32),
                pltpu.VMEM_SHARED((NPAD, dc), jnp.float32),
            ] + [pltpu.SemaphoreType.DMA] * (3 * NBP),
        )

    return deg, make_agg(2, D // 2), make_agg(1, 2 * DOUT)


def _norms(deg):
    dego = deg[0, :, 0:1] + deg[0, :, 1:2]
    degi = deg[1, :, 0:1] + deg[1, :, 1:2]
    no = lax.rsqrt(jnp.maximum(dego, 1.0))
    ni = lax.rsqrt(jnp.maximum(degi, 1.0))
    return no, ni


def _mm_a_body(x_ref, deg_ref, w_ref, oa_ref, ob_ref):
    no, _ = _norms(deg_ref[...])
    o = (x_ref[...] * no) @ w_ref[...]
    oa_ref[...] = o[:, : D // 2]
    ob_ref[...] = o[:, D // 2 :]


def _sum_cat(aa, ab):
    a = aa[0] + aa[1]
    b = ab[0] + ab[1]
    return jnp.concatenate([a, b], axis=1)


def _mm_b_body(agga_ref, aggb_ref, deg_ref, w_ref, oa_ref, ob_ref):
    no, ni = _norms(deg_ref[...])
    h = jnp.maximum(_sum_cat(agga_ref[...], aggb_ref[...]), 0.0) * (ni * no)
    o = h @ w_ref[...]
    oa_ref[...] = o[:, : D // 2]
    ob_ref[...] = o[:, D // 2 :]


def _mm_c_body(agga_ref, aggb_ref, noise_ref, deg_ref, w_ref, o_ref):
    no, ni = _norms(deg_ref[...])
    h2s = jnp.maximum(_sum_cat(agga_ref[...], aggb_ref[...]), 0.0) * (ni * no)
    w = w_ref[...]
    p = h2s @ w
    q = (noise_ref[...] * no) @ w
    o_ref[...] = jnp.concatenate([p + q, p], axis=1)


def _fin_body(aggz_ref, deg_ref, o_ref):
    _, ni = _norms(deg_ref[...])
    a = aggz_ref[...]
    o_ref[...] = (a[0] + a[1]) * ni


_half = jax.ShapeDtypeStruct((NPAD, D // 2), jnp.float32)
_mm_a = pl.pallas_call(_mm_a_body, out_shape=(_half, _half))
_mm_b = pl.pallas_call(_mm_b_body, out_shape=(_half, _half))
_mm_c = pl.pallas_call(
    _mm_c_body, out_shape=jax.ShapeDtypeStruct((NPAD, 2 * DOUT), jnp.float32)
)
_fin = pl.pallas_call(
    _fin_body, out_shape=jax.ShapeDtypeStruct((NPAD, 2 * DOUT), jnp.float32)
)


def kernel(features, noise, edge_index, noise_d, W0, W1, W2):
    f32 = jnp.float32
    del noise_d

    pad = jnp.full((EPAD - E,), N, jnp.int32)
    src = jnp.concatenate([edge_index[0], pad])
    dst = jnp.concatenate([edge_index[1], pad])
    edges = jnp.stack([src, dst])
    edges4 = edges.reshape(2, TCHT, CHUNK).transpose(1, 0, 2)
    dedges = jnp.stack([src, dst + NPAD]).reshape(2, NW, TNCH, CHUNK)
    dedges = dedges.transpose(1, 0, 2, 3).reshape(NW, 2 * TNCH, CHUNK)

    zeros128 = jnp.zeros((NPAD, 128), f32)
    zeros32 = jnp.zeros((NPAD, 2 * DOUT), f32)
    zdeg = jnp.zeros((2 * NPAD,), f32)
    ones = jnp.ones((CHUNK,), f32)
    rowpad = jnp.zeros((NPAD - N, D), f32)
    xpad = jnp.concatenate([features, rowpad])
    noisep = jnp.concatenate([noise, rowpad])

    deg_kernel, agg64x2, agg32 = _sc_kernels()
    deg_parts = (
        deg_kernel(dedges, ones, zdeg).reshape(NC, 2, NPAD).transpose(1, 2, 0)
    )

    zeros64 = zeros128[:, : D // 2]
    u0a, u0b = _mm_a(xpad, deg_parts, W0)
    a0a, a0b = agg64x2(u0a, u0b, edges4, zeros64)
    u1a, u1b = _mm_b(a0a, a0b, deg_parts, W1)
    a1a, a1b = agg64x2(u1a, u1b, edges4, zeros64)
    z = _mm_c(a1a, a1b, noisep, deg_parts, W2)
    (az,) = agg32(z, edges4, zeros32)
    out = _fin(az, deg_parts)
    return out[:N]


# speedup vs baseline: 1.1786x; 1.1786x over previous
import functools

import jax
import jax.numpy as jnp
from jax import lax
from jax.experimental import pallas as pl
from jax.experimental.pallas import tpu as pltpu
from jax.experimental.pallas import tpu_sc as plsc

N = 10000
E = 320000
D = 128
DOUT = 16

NC = 2
NS = 16
NW = NC * NS
CHUNK = 128
TNCH = 80
EPT = TNCH * CHUNK
EPAD = NW * EPT
NR = 79
NPAD = NR * 128
STRIPE = NPAD // NS
DSTRIPE = 2 * NPAD // NS

def _deg_kernel_body(didx_hbm, ones_hbm, zeros_hbm, out_hbm, idx_v, ones_v, buf_v, acc_sh):
    c = lax.axis_index("c")
    s = lax.axis_index("s")
    wid = s * NC + c
    pltpu.sync_copy(didx_hbm.at[wid], idx_v)
    stripe = pl.ds(s * DSTRIPE, DSTRIPE)
    pltpu.sync_copy(zeros_hbm.at[stripe], buf_v)
    pltpu.sync_copy(buf_v, acc_sh.at[stripe])
    plsc.subcore_barrier()

    pltpu.sync_copy(ones_hbm, ones_v)

    @pl.loop(0, 2 * TNCH)
    def _(j):
        pltpu.sync_copy(ones_v, acc_sh.at[idx_v.at[j]], add=True)

    plsc.subcore_barrier()
    pltpu.sync_copy(acc_sh.at[stripe], buf_v)
    pltpu.sync_copy(buf_v, out_hbm.at[pl.ds(c * 2 * NPAD + s * DSTRIPE, DSTRIPE)])


NBP = 8
CNT0 = 128
CNT1 = 32
TCHT = EPAD // CHUNK
C0TOT = NS * CNT0


def _ring(u_hbm, echunks_hbm, acc_sh, rows_v, slot_v, isem, gsem, ssem, start, cnt):
    for b in range(NBP):
        pltpu.async_copy(echunks_hbm.at[start + b], slot_v.at[b], isem[b])

    @pl.loop(0, cnt // NBP)
    def _(g):
        j0 = start + g * NBP
        for b in range(NBP):
            j = j0 + b
            pltpu.make_async_copy(echunks_hbm.at[j], slot_v.at[b], isem[b]).wait()
            pltpu.async_copy(u_hbm.at[slot_v.at[b, 0]], rows_v.at[b], gsem[b])
        for b in range(NBP):
            j = j0 + b
            pltpu.make_async_copy(u_hbm.at[slot_v.at[b, 0]], rows_v.at[b], gsem[b]).wait()
            pltpu.async_copy(rows_v.at[b], acc_sh.at[slot_v.at[b, 1]], ssem[b], add=True)
        for b in range(NBP):
            j = j0 + b
            jn = j + NBP
            pltpu.make_async_copy(rows_v.at[b], acc_sh.at[slot_v.at[b, 1]], ssem[b]).wait()

            @pl.when(jn < start + cnt)
            def _():
                pltpu.async_copy(echunks_hbm.at[jn], slot_v.at[b], isem[b])


def _aggp_body(nsplit, dc, *refs):
    us = refs[:nsplit]
    echunks_hbm = refs[nsplit]
    zeros_hbm = refs[nsplit + 1]
    outs = refs[nsplit + 2 : 2 * nsplit + 2]
    slot_v, rows_v, acc_sh = refs[2 * nsplit + 2 : 2 * nsplit + 5]
    sems = refs[2 * nsplit + 5 :]
    isem = sems[:NBP]
    gsem = sems[NBP : 2 * NBP]
    ssem = sems[2 * NBP :]
    c = lax.axis_index("c")
    s = lax.axis_index("s")
    stripe = pl.ds(s * STRIPE, STRIPE)

    for p in range(nsplit):
        u_hbm = us[p]
        pltpu.sync_copy(zeros_hbm.at[stripe], acc_sh.at[stripe])
        plsc.subcore_barrier()

        @pl.when(c == 0)
        def _():
            _ring(u_hbm, echunks_hbm, acc_sh, rows_v, slot_v,
                  isem, gsem, ssem, s * CNT0, CNT0)

        @pl.when(c == 1)
        def _():
            _ring(u_hbm, echunks_hbm, acc_sh, rows_v, slot_v,
                  isem, gsem, ssem, C0TOT + s * CNT1, CNT1)

        plsc.subcore_barrier()
        pltpu.sync_copy(acc_sh.at[stripe], outs[p].at[c, stripe])
        if p + 1 < nsplit:
            plsc.subcore_barrier()


@functools.lru_cache(maxsize=None)
def _sc_kernels(interpret=False):
    mesh = plsc.VectorSubcoreMesh(
        core_axis_name="c", subcore_axis_name="s", num_cores=NC, num_subcores=NS
    )
    deg = pl.kernel(
        _deg_kernel_body,
        out_type=jax.ShapeDtypeStruct((NC * 2 * NPAD,), jnp.float32),
        mesh=mesh,
        interpret=interpret,
        scratch_types=[
            pltpu.VMEM((2 * TNCH, CHUNK), jnp.int32),
            pltpu.VMEM((CHUNK,), jnp.float32),
            pltpu.VMEM((DSTRIPE,), jnp.float32),
            pltpu.VMEM_SHARED((2 * NPAD,), jnp.float32),
        ],
    )

    def make_agg(nsplit, dc):
        return pl.kernel(
            functools.partial(_aggp_body, nsplit, dc),
            out_type=[jax.ShapeDtypeStruct((NC, NPAD, dc), jnp.float32)] * nsplit,
            mesh=mesh,
            interpret=interpret,
            compiler_params=pltpu.CompilerParams(use_tc_tiling_on_sc=False),
            scratch_types=[
                pltpu.VMEM((NBP, 2, CHUNK), jnp.int32),
                pltpu.VMEM((NBP, CHUNK, dc), jnp.float32),
                pltpu.VMEM_SHARED((NPAD, dc), jnp.float32),
            ] + [pltpu.SemaphoreType.DMA] * (3 * NBP),
        )

    return deg, make_agg(2, D // 2), make_agg(1, 2 * DOUT)


def _norms(deg):
    dego = deg[0, :, 0:1] + deg[0, :, 1:2]
    degi = deg[1, :, 0:1] + deg[1, :, 1:2]
    no = lax.rsqrt(jnp.maximum(dego, 1.0))
    ni = lax.rsqrt(jnp.maximum(degi, 1.0))
    return no, ni


def _mm_a_body(x_ref, deg_ref, w_ref, oa_ref, ob_ref):
    no, _ = _norms(deg_ref[...])
    o = (x_ref[...] * no) @ w_ref[...]
    oa_ref[...] = o[:, : D // 2]
    ob_ref[...] = o[:, D // 2 :]


def _sum_cat(aa, ab):
    a = aa[0] + aa[1]
    b = ab[0] + ab[1]
    return jnp.concatenate([a, b], axis=1)


def _mm_b_body(agga_ref, aggb_ref, deg_ref, w_ref, oa_ref, ob_ref):
    no, ni = _norms(deg_ref[...])
    h = jnp.maximum(_sum_cat(agga_ref[...], aggb_ref[...]), 0.0) * (ni * no)
    o = h @ w_ref[...]
    oa_ref[...] = o[:, : D // 2]
    ob_ref[...] = o[:, D // 2 :]


def _mm_c_body(agga_ref, aggb_ref, noise_ref, deg_ref, w_ref, o_ref):
    no, ni = _norms(deg_ref[...])
    h2s = jnp.maximum(_sum_cat(agga_ref[...], aggb_ref[...]), 0.0) * (ni * no)
    w = w_ref[...]
    p = h2s @ w
    q = (noise_ref[...] * no) @ w
    o_ref[...] = jnp.concatenate([p + q, p], axis=1)


def _fin_body(aggz_ref, deg_ref, o_ref):
    _, ni = _norms(deg_ref[...])
    a = aggz_ref[...]
    o_ref[...] = (a[0] + a[1]) * ni


_half = jax.ShapeDtypeStruct((NPAD, D // 2), jnp.float32)
_mm_a = pl.pallas_call(_mm_a_body, out_shape=(_half, _half))
_mm_b = pl.pallas_call(_mm_b_body, out_shape=(_half, _half))
_mm_c = pl.pallas_call(
    _mm_c_body, out_shape=jax.ShapeDtypeStruct((NPAD, 2 * DOUT), jnp.float32)
)
_fin = pl.pallas_call(
    _fin_body, out_shape=jax.ShapeDtypeStruct((NPAD, 2 * DOUT), jnp.float32)
)


def kernel(features, noise, edge_index, noise_d, W0, W1, W2):
    f32 = jnp.float32
    del noise_d

    pad = jnp.full((EPAD - E,), N, jnp.int32)
    src = jnp.concatenate([edge_index[0], pad])
    dst = jnp.concatenate([edge_index[1], pad])
    edges = jnp.stack([src, dst])
    edges4 = edges.reshape(2, TCHT, CHUNK).transpose(1, 0, 2)
    dedges = jnp.stack([src, dst + NPAD]).reshape(2, NW, TNCH, CHUNK)
    dedges = dedges.transpose(1, 0, 2, 3).reshape(NW, 2 * TNCH, CHUNK)

    zeros128 = jnp.zeros((NPAD, 128), f32)
    zeros32 = jnp.zeros((NPAD, 2 * DOUT), f32)
    zdeg = jnp.zeros((2 * NPAD,), f32)
    ones = jnp.ones((CHUNK,), f32)
    rowpad = jnp.zeros((NPAD - N, D), f32)
    xpad = jnp.concatenate([features, rowpad])
    noisep = jnp.concatenate([noise, rowpad])

    deg_kernel, agg64x2, agg32 = _sc_kernels()
    deg_parts = (
        deg_kernel(dedges, ones, zdeg).reshape(NC, 2, NPAD).transpose(1, 2, 0)
    )

    zeros64 = zeros128[:, : D // 2]
    u0a, u0b = _mm_a(xpad, deg_parts, W0)
    a0a, a0b = agg64x2(u0a, u0b, edges4, zeros64)
    u1a, u1b = _mm_b(a0a, a0b, deg_parts, W1)
    a1a, a1b = agg64x2(u1a, u1b, edges4, zeros64)
    z = _mm_c(a1a, a1b, noisep, deg_parts, W2)
    (az,) = agg32(z, edges4, zeros32)
    out = _fin(az, deg_parts)
    return out[:N]


# speedup vs baseline: 1.1858x; 1.0061x over previous
import functools

import jax
import jax.numpy as jnp
from jax import lax
from jax.experimental import pallas as pl
from jax.experimental.pallas import tpu as pltpu
from jax.experimental.pallas import tpu_sc as plsc

N = 10000
E = 320000
D = 128
DOUT = 16

NC = 2
NS = 16
NW = NC * NS
CHUNK = 128
TNCH = 80
EPT = TNCH * CHUNK
EPAD = NW * EPT
NR = 79
NPAD = NR * 128
STRIPE = NPAD // NS
DSTRIPE = 2 * NPAD // NS

def _deg_kernel_body(didx_hbm, ones_hbm, zeros_hbm, out_hbm, idx_v, ones_v, buf_v, acc_sh):
    c = lax.axis_index("c")
    s = lax.axis_index("s")
    wid = s * NC + c
    pltpu.sync_copy(didx_hbm.at[wid], idx_v)
    stripe = pl.ds(s * DSTRIPE, DSTRIPE)
    pltpu.sync_copy(zeros_hbm.at[stripe], buf_v)
    pltpu.sync_copy(buf_v, acc_sh.at[stripe])
    plsc.subcore_barrier()

    pltpu.sync_copy(ones_hbm, ones_v)

    @pl.loop(0, 2 * TNCH)
    def _(j):
        pltpu.sync_copy(ones_v, acc_sh.at[idx_v.at[j]], add=True)

    plsc.subcore_barrier()
    pltpu.sync_copy(acc_sh.at[stripe], buf_v)
    pltpu.sync_copy(buf_v, out_hbm.at[pl.ds(c * 2 * NPAD + s * DSTRIPE, DSTRIPE)])


NBP = 4
CNT0 = 128
CNT1 = 32
TCHT = EPAD // CHUNK
C0TOT = NS * CNT0


def _ring(u_hbm, echunks_hbm, acc_sh, rows_v, slot_v, isem, gsem, ssem, start, cnt):
    ng = cnt // NBP
    last = ng - 1

    def idx_fetch(g, par, b):
        pltpu.async_copy(
            echunks_hbm.at[start + g * NBP + b], slot_v.at[par, b], isem[par][b]
        )

    def idx_wait(g, par, b):
        pltpu.make_async_copy(
            echunks_hbm.at[start + g * NBP + b], slot_v.at[par, b], isem[par][b]
        ).wait()

    def gather(g, par, b, wait):
        args = (u_hbm.at[slot_v.at[par, b, 0]], rows_v.at[b], gsem[b])
        if wait:
            pltpu.make_async_copy(*args).wait()
        else:
            pltpu.async_copy(*args)

    def scatter(g, par, b, wait):
        args = (rows_v.at[b], acc_sh.at[slot_v.at[par, b, 1]], ssem[b])
        if wait:
            pltpu.make_async_copy(*args).wait()
        else:
            pltpu.async_copy(*args, add=True)

    for b in range(NBP):
        idx_fetch(0, 0, b)
    if ng > 1:
        for b in range(NBP):
            idx_fetch(1, 1, b)
    for b in range(NBP):
        idx_wait(0, 0, b)
        gather(0, 0, b, wait=False)

    @pl.loop(0, ng // 2)
    def _(i):
        for par in range(2):
            g = 2 * i + par
            for b in range(NBP):
                gather(g, par, b, wait=True)
                scatter(g, par, b, wait=False)
            for b in range(NBP):
                scatter(g, par, b, wait=True)

                @pl.when(g + 2 <= last)
                def _():
                    idx_fetch(g + 2, par, b)

                @pl.when(g + 1 <= last)
                def _():
                    idx_wait(g + 1, 1 - par, b)
                    gather(g + 1, 1 - par, b, wait=False)


def _aggp_body(nsplit, dc, *refs):
    us = refs[:nsplit]
    echunks_hbm = refs[nsplit]
    zeros_hbm = refs[nsplit + 1]
    outs = refs[nsplit + 2 : 2 * nsplit + 2]
    slot_v, rows_v, acc_sh = refs[2 * nsplit + 2 : 2 * nsplit + 5]
    sems = refs[2 * nsplit + 5 :]
    isem = (sems[:NBP], sems[NBP : 2 * NBP])
    gsem = sems[2 * NBP : 3 * NBP]
    ssem = sems[3 * NBP :]
    c = lax.axis_index("c")
    s = lax.axis_index("s")
    stripe = pl.ds(s * STRIPE, STRIPE)

    for p in range(nsplit):
        u_hbm = us[p]
        pltpu.sync_copy(zeros_hbm.at[stripe], acc_sh.at[stripe])
        plsc.subcore_barrier()

        @pl.when(c == 0)
        def _():
            _ring(u_hbm, echunks_hbm, acc_sh, rows_v, slot_v,
                  isem, gsem, ssem, s * CNT0, CNT0)

        @pl.when(c == 1)
        def _():
            _ring(u_hbm, echunks_hbm, acc_sh, rows_v, slot_v,
                  isem, gsem, ssem, C0TOT + s * CNT1, CNT1)

        plsc.subcore_barrier()
        pltpu.sync_copy(acc_sh.at[stripe], outs[p].at[c, stripe])
        if p + 1 < nsplit:
            plsc.subcore_barrier()


@functools.lru_cache(maxsize=None)
def _sc_kernels(interpret=False):
    mesh = plsc.VectorSubcoreMesh(
        core_axis_name="c", subcore_axis_name="s", num_cores=NC, num_subcores=NS
    )
    deg = pl.kernel(
        _deg_kernel_body,
        out_type=jax.ShapeDtypeStruct((NC * 2 * NPAD,), jnp.float32),
        mesh=mesh,
        interpret=interpret,
        scratch_types=[
            pltpu.VMEM((2 * TNCH, CHUNK), jnp.int32),
            pltpu.VMEM((CHUNK,), jnp.float32),
            pltpu.VMEM((DSTRIPE,), jnp.float32),
            pltpu.VMEM_SHARED((2 * NPAD,), jnp.float32),
        ],
    )

    def make_agg(nsplit, dc):
        return pl.kernel(
            functools.partial(_aggp_body, nsplit, dc),
            out_type=[jax.ShapeDtypeStruct((NC, NPAD, dc), jnp.float32)] * nsplit,
            mesh=mesh,
            interpret=interpret,
            compiler_params=pltpu.CompilerParams(use_tc_tiling_on_sc=False),
            scratch_types=[
                pltpu.VMEM((2, NBP, 2, CHUNK), jnp.int32),
                pltpu.VMEM((NBP, CHUNK, dc), jnp.float32),
                pltpu.VMEM_SHARED((NPAD, dc), jnp.float32),
            ] + [pltpu.SemaphoreType.DMA] * (4 * NBP),
        )

    return deg, make_agg(2, D // 2), make_agg(1, 2 * DOUT)


def _norms(deg):
    dego = deg[0, :, 0:1] + deg[0, :, 1:2]
    degi = deg[1, :, 0:1] + deg[1, :, 1:2]
    no = lax.rsqrt(jnp.maximum(dego, 1.0))
    ni = lax.rsqrt(jnp.maximum(degi, 1.0))
    return no, ni


def _mm_a_body(x_ref, deg_ref, w_ref, oa_ref, ob_ref):
    no, _ = _norms(deg_ref[...])
    o = (x_ref[...] * no) @ w_ref[...]
    oa_ref[...] = o[:, : D // 2]
    ob_ref[...] = o[:, D // 2 :]


def _sum_cat(aa, ab):
    a = aa[0] + aa[1]
    b = ab[0] + ab[1]
    return jnp.concatenate([a, b], axis=1)


def _mm_b_body(agga_ref, aggb_ref, deg_ref, w_ref, oa_ref, ob_ref):
    no, ni = _norms(deg_ref[...])
    h = jnp.maximum(_sum_cat(agga_ref[...], aggb_ref[...]), 0.0) * (ni * no)
    o = h @ w_ref[...]
    oa_ref[...] = o[:, : D // 2]
    ob_ref[...] = o[:, D // 2 :]


def _mm_c_body(agga_ref, aggb_ref, noise_ref, deg_ref, w_ref, o_ref):
    no, ni = _norms(deg_ref[...])
    h2s = jnp.maximum(_sum_cat(agga_ref[...], aggb_ref[...]), 0.0) * (ni * no)
    w = w_ref[...]
    p = h2s @ w
    q = (noise_ref[...] * no) @ w
    o_ref[...] = jnp.concatenate([p + q, p], axis=1)


def _fin_body(aggz_ref, deg_ref, o_ref):
    _, ni = _norms(deg_ref[...])
    a = aggz_ref[...]
    o_ref[...] = (a[0] + a[1]) * ni


_half = jax.ShapeDtypeStruct((NPAD, D // 2), jnp.float32)
_mm_a = pl.pallas_call(_mm_a_body, out_shape=(_half, _half))
_mm_b = pl.pallas_call(_mm_b_body, out_shape=(_half, _half))
_mm_c = pl.pallas_call(
    _mm_c_body, out_shape=jax.ShapeDtypeStruct((NPAD, 2 * DOUT), jnp.float32)
)
_fin = pl.pallas_call(
    _fin_body, out_shape=jax.ShapeDtypeStruct((NPAD, 2 * DOUT), jnp.float32)
)


def kernel(features, noise, edge_index, noise_d, W0, W1, W2):
    f32 = jnp.float32
    del noise_d

    pad = jnp.full((EPAD - E,), N, jnp.int32)
    src = jnp.concatenate([edge_index[0], pad])
    dst = jnp.concatenate([edge_index[1], pad])
    edges = jnp.stack([src, dst])
    edges4 = edges.reshape(2, TCHT, CHUNK).transpose(1, 0, 2)
    dedges = jnp.stack([src, dst + NPAD]).reshape(2, NW, TNCH, CHUNK)
    dedges = dedges.transpose(1, 0, 2, 3).reshape(NW, 2 * TNCH, CHUNK)

    zeros128 = jnp.zeros((NPAD, 128), f32)
    zeros32 = jnp.zeros((NPAD, 2 * DOUT), f32)
    zdeg = jnp.zeros((2 * NPAD,), f32)
    ones = jnp.ones((CHUNK,), f32)
    rowpad = jnp.zeros((NPAD - N, D), f32)
    xpad = jnp.concatenate([features, rowpad])
    noisep = jnp.concatenate([noise, rowpad])

    deg_kernel, agg64x2, agg32 = _sc_kernels()
    deg_parts = (
        deg_kernel(dedges, ones, zdeg).reshape(NC, 2, NPAD).transpose(1, 2, 0)
    )

    zeros64 = zeros128[:, : D // 2]
    u0a, u0b = _mm_a(xpad, deg_parts, W0)
    a0a, a0b = agg64x2(u0a, u0b, edges4, zeros64)
    u1a, u1b = _mm_b(a0a, a0b, deg_parts, W1)
    a1a, a1b = agg64x2(u1a, u1b, edges4, zeros64)
    z = _mm_c(a1a, a1b, noisep, deg_parts, W2)
    (az,) = agg32(z, edges4, zeros32)
    out = _fin(az, deg_parts)
    return out[:N]
